# baseline (speedup 1.0000x reference)
import functools

import jax
import jax.numpy as jnp
from jax import lax
from jax.experimental import pallas as pl
from jax.experimental.pallas import tpu as pltpu
from jax.experimental.pallas import tpu_sc as plsc

D_MODEL = 64
SAMPLE = 200
BATCH = 4096
LANES = 16
NUM_CORES = 2
NUM_SUBCORES = 16
NUM_WORKERS = NUM_CORES * NUM_SUBCORES
NB = BATCH // NUM_WORKERS
CHUNK = 100
GROUPS = 13
ROWS = SAMPLE + 8
UNROLL = 4


def _sc_body(h_hbm, idx_hbm, tbl_hbm, out_hbm,
             h_v, idx_v, di0, di1, rows0, rows1, out_st, sg0, sg1):
    wid = lax.axis_index("s") * NUM_CORES + lax.axis_index("c")
    b0 = wid * NB

    pltpu.sync_copy(h_hbm.at[pl.ds(b0 * D_MODEL, NB * D_MODEL)], h_v)
    pltpu.sync_copy(idx_hbm.at[pl.ds(wid * (2 * NB), 2 * NB)], idx_v)

    iota = lax.iota(jnp.int32, LANES)
    rowidx = [iota + LANES * g for g in range(GROUPS)]
    crow = []
    ccol = []
    for g in range(GROUPS):
        s = LANES * g + jnp.arange(LANES, dtype=jnp.int32)
        r = (s >= CHUNK).astype(jnp.int32)
        crow.append(jnp.asarray(r))
        ccol.append(jnp.asarray(s - CHUNK * r))

    def shift_idx(bl, di):
        for c in range(2):
            src = idx_v.at[2 * bl + c]
            for off in (0, 16, 32, 48, 64, 80, 84):
                di[c, pl.ds(off, LANES)] = src[pl.ds(off, LANES)] >> 1

    def gather_cps(di, rows, sem):
        return [
            pltpu.make_async_copy(
                tbl_hbm.at[di.at[c]],
                rows.at[pl.ds(CHUNK * c, CHUNK)],
                sem,
            )
            for c in range(2)
        ]

    def fire(bl, di, rows, sem):
        shift_idx(bl, di)
        for cp in gather_cps(di, rows, sem):
            cp.start()

    def wait(di, rows, sem):
        for cp in gather_cps(di, rows, sem):
            cp.wait()

    def compute(bl, rows):
        hbase = bl * D_MODEL
        coloff = [
            (plsc.load_gather(idx_v, [2 * bl + crow[g], ccol[g]]) & 1) << 6
            for g in range(GROUPS)
        ]

        def dbody(i, accs):
            for k in range(UNROLL):
                d = i * UNROLL + k
                skew = (iota + d) & (D_MODEL - 1)
                hb = plsc.load_gather(h_v, [hbase + skew])
                accs = tuple(
                    acc + hb * plsc.load_gather(rows, [rowidx[g], coloff[g] + skew])
                    for g, acc in enumerate(accs)
                )
            return accs

        zero = jnp.zeros((LANES,), jnp.float32)
        accs = lax.fori_loop(0, D_MODEL // UNROLL, dbody, (zero,) * GROUPS)

        obase = bl * SAMPLE
        for g in range(GROUPS):
            out_st[pl.ds(obase + LANES * g, LANES)] = accs[g]

    fire(0, di0, rows0, sg0)

    def pair(i, carry):
        a = 2 * i
        fire(a + 1, di1, rows1, sg1)
        wait(di0, rows0, sg0)
        compute(a, rows0)

        @pl.when(i < NB // 2 - 1)
        def _():
            fire(a + 2, di0, rows0, sg0)

        wait(di1, rows1, sg1)
        compute(a + 1, rows1)
        return carry

    lax.fori_loop(0, NB // 2, pair, 0)

    pltpu.sync_copy(
        out_st.at[pl.ds(0, NB * SAMPLE)],
        out_hbm.at[pl.ds(wid * NB * SAMPLE, NB * SAMPLE)],
    )


@jax.jit
def _embedding_dot(h2, idx2, table2):
    mesh = plsc.VectorSubcoreMesh(
        core_axis_name="c", subcore_axis_name="s",
        num_cores=NUM_CORES, num_subcores=NUM_SUBCORES,
    )
    call = functools.partial(
        pl.kernel,
        out_type=jax.ShapeDtypeStruct((BATCH * SAMPLE,), jnp.float32),
        mesh=mesh,
        scratch_types=[
            pltpu.VMEM((NB * D_MODEL,), jnp.float32),
            pltpu.VMEM((2 * NB, CHUNK), jnp.int32),
            pltpu.VMEM((2, CHUNK), jnp.int32),
            pltpu.VMEM((2, CHUNK), jnp.int32),
            pltpu.VMEM((ROWS, 2 * D_MODEL), jnp.float32),
            pltpu.VMEM((ROWS, 2 * D_MODEL), jnp.float32),
            pltpu.VMEM((NB * SAMPLE + 8,), jnp.float32),
            pltpu.SemaphoreType.DMA,
            pltpu.SemaphoreType.DMA,
        ],
        compiler_params=pltpu.CompilerParams(
            needs_layout_passes=False, use_tc_tiling_on_sc=False
        ),
    )
    return call(_sc_body)(h2, idx2, table2)


def kernel(h, indicies, embedding_weight):
    b, s = indicies.shape
    n, d = embedding_weight.shape
    h2 = jnp.reshape(h, (b * D_MODEL,))
    idx2 = jnp.reshape(indicies.astype(jnp.int32), (2 * b, CHUNK))
    table2 = jnp.reshape(embedding_weight, (n // 2, 2 * d))
    out = _embedding_dot(h2, idx2, table2)
    return jnp.reshape(out, (b, 1, s))


# speedup vs baseline: 1.0615x; 1.0615x over previous
import functools

import jax
import jax.numpy as jnp
from jax import lax
from jax.experimental import pallas as pl
from jax.experimental.pallas import tpu as pltpu
from jax.experimental.pallas import tpu_sc as plsc

D_MODEL = 64
SAMPLE = 200
BATCH = 4096
LANES = 16
NUM_CORES = 2
NUM_SUBCORES = 16
NUM_WORKERS = NUM_CORES * NUM_SUBCORES
NB = BATCH // NUM_WORKERS
CHUNK = 100
GROUPS = 13
ROWS = SAMPLE + 16
UNROLL = 4


def _sc_body(h_hbm, idx_hbm, tbl_hbm, out_hbm,
             h_v, idx_v, di0, di1, rows0, rows1, out_st, sg0, sg1):
    wid = lax.axis_index("s") * NUM_CORES + lax.axis_index("c")
    b0 = wid * NB

    pltpu.sync_copy(h_hbm.at[pl.ds(b0 * D_MODEL, NB * D_MODEL)], h_v)
    pltpu.sync_copy(idx_hbm.at[pl.ds(wid * (2 * NB), 2 * NB)], idx_v)

    iota = lax.iota(jnp.int32, LANES)
    rowidx = []
    for g in range(GROUPS):
        s = LANES * g + jnp.arange(LANES, dtype=jnp.int32)
        rowidx.append(jnp.asarray(s + 4 * (s >= CHUNK)))
    crow = []
    ccol = []
    for g in range(GROUPS):
        s = LANES * g + jnp.arange(LANES, dtype=jnp.int32)
        r = (s >= CHUNK).astype(jnp.int32)
        crow.append(jnp.asarray(r))
        ccol.append(jnp.asarray(s - CHUNK * r))

    def shift_idx(bl, di):
        for c in range(2):
            src = idx_v.at[2 * bl + c]
            for off in (0, 16, 32, 48, 64, 80, 84):
                di[c, pl.ds(off, LANES)] = src[pl.ds(off, LANES)] >> 1

    def gather_cps(di, rows, sem):
        return [
            pltpu.make_async_copy(
                tbl_hbm.at[di.at[c]],
                rows.at[pl.ds(104 * c, CHUNK)],
                sem,
            )
            for c in range(2)
        ]

    def fire(bl, di, rows, sem):
        shift_idx(bl, di)
        for cp in gather_cps(di, rows, sem):
            cp.start()

    def wait(di, rows, sem):
        for cp in gather_cps(di, rows, sem):
            cp.wait()

    def compute(bl, rows):
        hbase = bl * D_MODEL
        coloff = [
            (plsc.load_gather(idx_v, [2 * bl + crow[g], ccol[g]]) & 1) << 6
            for g in range(GROUPS)
        ]

        def dbody(i, accs):
            for k in range(UNROLL):
                d = i * UNROLL + k
                skew = (iota + d) & (D_MODEL - 1)
                hb = plsc.load_gather(h_v, [hbase + skew])
                accs = tuple(
                    acc + hb * plsc.load_gather(rows, [rowidx[g], coloff[g] + skew])
                    for g, acc in enumerate(accs)
                )
            return accs

        zero = jnp.zeros((LANES,), jnp.float32)
        accs = lax.fori_loop(0, D_MODEL // UNROLL, dbody, (zero,) * GROUPS)

        obase = bl * SAMPLE
        for g in range(GROUPS):
            out_st[pl.ds(obase + LANES * g, LANES)] = accs[g]

    fire(0, di0, rows0, sg0)

    def pair(i, carry):
        a = 2 * i
        fire(a + 1, di1, rows1, sg1)
        wait(di0, rows0, sg0)
        compute(a, rows0)

        @pl.when(i < NB // 2 - 1)
        def _():
            fire(a + 2, di0, rows0, sg0)

        wait(di1, rows1, sg1)
        compute(a + 1, rows1)
        return carry

    lax.fori_loop(0, NB // 2, pair, 0)

    pltpu.sync_copy(
        out_st.at[pl.ds(0, NB * SAMPLE)],
        out_hbm.at[pl.ds(wid * NB * SAMPLE, NB * SAMPLE)],
    )


@jax.jit
def _embedding_dot(h2, idx2, table2):
    mesh = plsc.VectorSubcoreMesh(
        core_axis_name="c", subcore_axis_name="s",
        num_cores=NUM_CORES, num_subcores=NUM_SUBCORES,
    )
    call = functools.partial(
        pl.kernel,
        out_type=jax.ShapeDtypeStruct((BATCH * SAMPLE,), jnp.float32),
        mesh=mesh,
        scratch_types=[
            pltpu.VMEM((NB * D_MODEL,), jnp.float32),
            pltpu.VMEM((2 * NB, CHUNK), jnp.int32),
            pltpu.VMEM((2, CHUNK), jnp.int32),
            pltpu.VMEM((2, CHUNK), jnp.int32),
            pltpu.VMEM((ROWS, 2 * D_MODEL), jnp.float32),
            pltpu.VMEM((ROWS, 2 * D_MODEL), jnp.float32),
            pltpu.VMEM((NB * SAMPLE + 8,), jnp.float32),
            pltpu.SemaphoreType.DMA,
            pltpu.SemaphoreType.DMA,
        ],
        compiler_params=pltpu.CompilerParams(
            needs_layout_passes=False, use_tc_tiling_on_sc=True
        ),
    )
    return call(_sc_body)(h2, idx2, table2)


def kernel(h, indicies, embedding_weight):
    b, s = indicies.shape
    n, d = embedding_weight.shape
    h2 = jnp.reshape(h, (b * D_MODEL,))
    idx2 = jnp.reshape(indicies.astype(jnp.int32), (2 * b, CHUNK))
    table2 = jnp.reshape(embedding_weight, (n // 2, 2 * d))
    out = _embedding_dot(h2, idx2, table2)
    return jnp.reshape(out, (b, 1, s))


# speedup vs baseline: 1.1285x; 1.0631x over previous
import functools

import jax
import jax.numpy as jnp
from jax import lax
from jax.experimental import pallas as pl
from jax.experimental.pallas import tpu as pltpu
from jax.experimental.pallas import tpu_sc as plsc

D_MODEL = 64
SAMPLE = 200
BATCH = 4096
LANES = 16
NUM_CORES = 2
NUM_SUBCORES = 16
NUM_WORKERS = NUM_CORES * NUM_SUBCORES
NB = BATCH // NUM_WORKERS
CHUNK = 100
GROUPS = 13
BPB = 2
ROWS = BPB * SAMPLE + 8
UNROLL = 4


def _sc_body(h_hbm, idx_hbm, tbl_hbm, out_hbm,
             h_v, idx_v, rows0, rows1, out_st, sg0, sg1):
    wid = lax.axis_index("s") * NUM_CORES + lax.axis_index("c")
    b0 = wid * NB

    pltpu.sync_copy(h_hbm.at[pl.ds(b0 * D_MODEL, NB * D_MODEL)], h_v)
    pltpu.sync_copy(idx_hbm.at[pl.ds(wid * (2 * NB), 2 * NB)], idx_v)

    iota = lax.iota(jnp.int32, LANES)
    rowidx = [
        [iota + slot * SAMPLE + LANES * g for g in range(GROUPS)]
        for slot in range(BPB)
    ]

    def gather_cps(b_first, rows, sem):
        return [
            pltpu.make_async_copy(
                tbl_hbm.at[idx_v.at[2 * b_first + c]],
                rows.at[pl.ds(CHUNK * c, CHUNK)],
                sem,
            )
            for c in range(2 * BPB)
        ]

    def fire(b_first, rows, sem):
        for cp in gather_cps(b_first, rows, sem):
            cp.start()

    def wait(b_first, rows, sem):
        for cp in gather_cps(b_first, rows, sem):
            cp.wait()

    def compute(bl, rows, slot):
        hbase = bl * D_MODEL

        def dbody(i, accs):
            for k in range(UNROLL):
                d = i * UNROLL + k
                colv = (iota + d) & (D_MODEL - 1)
                hb = plsc.load_gather(h_v, [hbase + colv])
                accs = tuple(
                    acc + hb * plsc.load_gather(rows, [rowidx[slot][g], colv])
                    for g, acc in enumerate(accs)
                )
            return accs

        zero = jnp.zeros((LANES,), jnp.float32)
        accs = lax.fori_loop(0, D_MODEL // UNROLL, dbody, (zero,) * GROUPS)

        obase = bl * SAMPLE
        for g in range(GROUPS):
            out_st[pl.ds(obase + LANES * g, LANES)] = accs[g]

    fire(0, rows0, sg0)

    def quad(i, carry):
        a = BPB * 2 * i
        fire(a + BPB, rows1, sg1)
        wait(a, rows0, sg0)
        compute(a + 0, rows0, 0)
        compute(a + 1, rows0, 1)

        @pl.when(i < NB // (2 * BPB) - 1)
        def _():
            fire(a + 2 * BPB, rows0, sg0)

        wait(a + BPB, rows1, sg1)
        compute(a + BPB + 0, rows1, 0)
        compute(a + BPB + 1, rows1, 1)
        return carry

    lax.fori_loop(0, NB // (2 * BPB), quad, 0)

    pltpu.sync_copy(
        out_st.at[pl.ds(0, NB * SAMPLE)],
        out_hbm.at[pl.ds(wid * NB * SAMPLE, NB * SAMPLE)],
    )


@jax.jit
def _embedding_dot(h2, idx2, table):
    mesh = plsc.VectorSubcoreMesh(
        core_axis_name="c", subcore_axis_name="s",
        num_cores=NUM_CORES, num_subcores=NUM_SUBCORES,
    )
    call = functools.partial(
        pl.kernel,
        out_type=jax.ShapeDtypeStruct((BATCH * SAMPLE,), jnp.float32),
        mesh=mesh,
        scratch_types=[
            pltpu.VMEM((NB * D_MODEL,), jnp.float32),
            pltpu.VMEM((2 * NB, CHUNK), jnp.int32),
            pltpu.VMEM((ROWS, D_MODEL), jnp.float32),
            pltpu.VMEM((ROWS, D_MODEL), jnp.float32),
            pltpu.VMEM((NB * SAMPLE + 8,), jnp.float32),
            pltpu.SemaphoreType.DMA,
            pltpu.SemaphoreType.DMA,
        ],
        compiler_params=pltpu.CompilerParams(
            needs_layout_passes=False, use_tc_tiling_on_sc=False
        ),
    )
    return call(_sc_body)(h2, idx2, table)


def kernel(h, indicies, embedding_weight):
    b, s = indicies.shape
    h2 = jnp.reshape(h, (b * D_MODEL,))
    idx2 = jnp.reshape(indicies.astype(jnp.int32), (2 * b, CHUNK))
    out = _embedding_dot(h2, idx2, embedding_weight)
    return jnp.reshape(out, (b, 1, s))


# speedup vs baseline: 1.1514x; 1.0203x over previous
import functools

import jax
import jax.numpy as jnp
from jax import lax
from jax.experimental import pallas as pl
from jax.experimental.pallas import tpu as pltpu
from jax.experimental.pallas import tpu_sc as plsc

D_MODEL = 64
SAMPLE = 200
BATCH = 4096
LANES = 16
NUM_CORES = 2
NUM_SUBCORES = 16
NUM_WORKERS = NUM_CORES * NUM_SUBCORES
NB = BATCH // NUM_WORKERS
CHUNK = 100
SEAM = 104
GROUPS = 13
ROWS = SAMPLE + 16
UNROLL = 4


def _sc_body(h_hbm, idx_hbm, tbl_hbm, out_hbm,
             h_v, idx_v, rows0, rows1, out_st, sg0, sg1):
    wid = lax.axis_index("s") * NUM_CORES + lax.axis_index("c")
    b0 = wid * NB

    pltpu.sync_copy(h_hbm.at[pl.ds(b0 * D_MODEL, NB * D_MODEL)], h_v)
    pltpu.sync_copy(idx_hbm.at[pl.ds(wid * (2 * NB), 2 * NB)], idx_v)

    iota = lax.iota(jnp.int32, LANES)
    rowidx = []
    for g in range(GROUPS):
        s = LANES * g + jnp.arange(LANES, dtype=jnp.int32)
        rowidx.append(jnp.asarray(s + (SEAM - CHUNK) * (s >= CHUNK)))

    def gather_cps(bl, rows, sem):
        return [
            pltpu.make_async_copy(
                tbl_hbm.at[idx_v.at[2 * bl + c]],
                rows.at[pl.ds(SEAM * c, CHUNK)],
                sem,
            )
            for c in range(2)
        ]

    def fire(bl, rows, sem):
        for cp in gather_cps(bl, rows, sem):
            cp.start()

    def wait(bl, rows, sem):
        for cp in gather_cps(bl, rows, sem):
            cp.wait()

    def compute(bl, rows):
        hbase = bl * D_MODEL

        def dbody(i, accs):
            for k in range(UNROLL):
                d = i * UNROLL + k
                colv = (iota + d) & (D_MODEL - 1)
                hb = plsc.load_gather(h_v, [hbase + colv])
                accs = tuple(
                    acc + hb * plsc.load_gather(rows, [rowidx[g], colv])
                    for g, acc in enumerate(accs)
                )
            return accs

        zero = jnp.zeros((LANES,), jnp.float32)
        accs = lax.fori_loop(0, D_MODEL // UNROLL, dbody, (zero,) * GROUPS)

        obase = bl * SAMPLE
        for g in range(GROUPS):
            out_st[pl.ds(obase + LANES * g, LANES)] = accs[g]

    fire(0, rows0, sg0)

    def pair(i, carry):
        a = 2 * i
        fire(a + 1, rows1, sg1)
        wait(a, rows0, sg0)
        compute(a, rows0)

        @pl.when(i < NB // 2 - 1)
        def _():
            fire(a + 2, rows0, sg0)

        wait(a + 1, rows1, sg1)
        compute(a + 1, rows1)
        return carry

    lax.fori_loop(0, NB // 2, pair, 0)

    pltpu.sync_copy(
        out_st.at[pl.ds(0, NB * SAMPLE)],
        out_hbm.at[pl.ds(wid * NB * SAMPLE, NB * SAMPLE)],
    )


@jax.jit
def _embedding_dot(h2, idx2, tablep):
    mesh = plsc.VectorSubcoreMesh(
        core_axis_name="c", subcore_axis_name="s",
        num_cores=NUM_CORES, num_subcores=NUM_SUBCORES,
    )
    call = functools.partial(
        pl.kernel,
        out_type=jax.ShapeDtypeStruct((BATCH * SAMPLE,), jnp.float32),
        mesh=mesh,
        scratch_types=[
            pltpu.VMEM((NB * D_MODEL,), jnp.float32),
            pltpu.VMEM((2 * NB, CHUNK), jnp.int32),
            pltpu.VMEM((ROWS, 2 * D_MODEL), jnp.float32),
            pltpu.VMEM((ROWS, 2 * D_MODEL), jnp.float32),
            pltpu.VMEM((NB * SAMPLE + 8,), jnp.float32),
            pltpu.SemaphoreType.DMA,
            pltpu.SemaphoreType.DMA,
        ],
        compiler_params=pltpu.CompilerParams(
            needs_layout_passes=False, use_tc_tiling_on_sc=True
        ),
    )
    return call(_sc_body)(h2, idx2, tablep)


def kernel(h, indicies, embedding_weight):
    b, s = indicies.shape
    h2 = jnp.reshape(h, (b * D_MODEL,))
    idx2 = jnp.reshape(indicies.astype(jnp.int32), (2 * b, CHUNK))
    tablep = jnp.pad(embedding_weight, ((0, 0), (0, D_MODEL)))
    out = _embedding_dot(h2, idx2, tablep)
    return jnp.reshape(out, (b, 1, s))
